# baseline (speedup 1.0000x reference)
import jax
import jax.numpy as jnp
from jax.experimental import pallas as pl
from jax.experimental.pallas import tpu as pltpu

_MINUTE_SIZE = 4
_HOUR_SIZE = 24
_WEEKDAY_SIZE = 7
_DAY_SIZE = 32
_MONTH_SIZE = 13

_SIZES = (_MONTH_SIZE, _DAY_SIZE, _WEEKDAY_SIZE, _HOUR_SIZE, _MINUTE_SIZE)
_BITS = (4, 5, 3, 5, 2)
_SHIFTS = (0, 4, 9, 12, 17)
_FUSED_ROWS = 128
_LANES = 128


def _sublane_tables():
    iota = jax.lax.broadcasted_iota(jnp.int32, (_FUSED_ROWS, 1), 0)
    mask = jnp.zeros((_FUSED_ROWS, 1), jnp.int32)
    tgt = jnp.ones((_FUSED_ROWS, 1), jnp.int32)
    off = 0
    for f, size in enumerate(_SIZES):
        in_f = jnp.logical_and(iota >= off, iota < off + size)
        mask = jnp.where(in_f, ((1 << _BITS[f]) - 1) << _SHIFTS[f], mask)
        tgt = jnp.where(in_f, (iota - off) << _SHIFTS[f], tgt)
        off += size
    return mask, tgt


def _make_body(n_groups):
    def _body(code_ref, tbl_ref, out_ref):
        tbl = tbl_ref[...]
        mask, tgt = _sublane_tables()

        def group(g, carry):
            g0 = pl.multiple_of(g, 1)
            code = code_ref[pl.ds(g0, 1), :]
            hot_t = ((code & mask) == tgt).astype(jnp.bfloat16)
            out_ref[pl.ds(g0 * _LANES, _LANES), :] = jax.lax.dot_general(
                hot_t, tbl, (((0,), (0,)), ((), ())),
                preferred_element_type=jnp.float32)
            return carry

        jax.lax.fori_loop(0, n_groups, group, None, unroll=True)

    return _body


def kernel(inputs, fused_table, *, tile_rows=4096):
    B, L, F = inputs.shape
    assert F == len(_SIZES)
    k_rows, d_model = fused_table.shape
    assert k_rows == _FUSED_ROWS

    N = B * L
    assert N % _LANES == 0
    x = inputs.astype(jnp.int32).reshape(N // _LANES, _LANES, F)
    code = (x[..., 0] | (x[..., 1] << _SHIFTS[1]) | (x[..., 2] << _SHIFTS[2])
            | (x[..., 3] << _SHIFTS[3]) | (x[..., 4] << _SHIFTS[4]))
    tbl16 = fused_table.astype(jnp.bfloat16)

    tile_rows = min(tile_rows, N)
    assert tile_rows % _LANES == 0 and N % tile_rows == 0
    n_groups = tile_rows // _LANES
    steps = N // tile_rows

    dp = ((d_model + 127) // 128) * 128
    vmem_need = (2 * n_groups * _LANES * 4
                 + 2 * tile_rows * dp * 4
                 + 2 * _FUSED_ROWS * dp * 2
                 + (4 << 20))
    vmem_limit = int(min(56 << 20, max(int(vmem_need * 1.2), 16 << 20)))

    out = pl.pallas_call(
        _make_body(n_groups),
        out_shape=jax.ShapeDtypeStruct((N, d_model), jnp.float32),
        grid=(steps,),
        in_specs=[
            pl.BlockSpec((n_groups, _LANES), lambda i: (i, 0)),
            pl.BlockSpec((_FUSED_ROWS, d_model), lambda i: (0, 0)),
        ],
        out_specs=pl.BlockSpec((tile_rows, d_model), lambda i: (i, 0)),
        compiler_params=pltpu.CompilerParams(
            dimension_semantics=("arbitrary",),
            vmem_limit_bytes=vmem_limit,
        ),
    )(code, tbl16)

    return out.reshape(B, L, d_model)


# speedup vs baseline: 1.0129x; 1.0129x over previous
import jax
import jax.numpy as jnp
from jax.experimental import pallas as pl
from jax.experimental.pallas import tpu as pltpu

_MINUTE_SIZE = 4
_HOUR_SIZE = 24
_WEEKDAY_SIZE = 7
_DAY_SIZE = 32
_MONTH_SIZE = 13

_SIZES = (_MONTH_SIZE, _DAY_SIZE, _WEEKDAY_SIZE, _HOUR_SIZE, _MINUTE_SIZE)
_BITS = (4, 5, 3, 5, 2)
_SHIFTS = (0, 4, 9, 12, 17)
_FUSED_ROWS = 128
_LANES = 128


def _sublane_tables():
    iota = jax.lax.broadcasted_iota(jnp.int32, (_FUSED_ROWS, 1), 0)
    mask = jnp.zeros((_FUSED_ROWS, 1), jnp.int32)
    tgt = jnp.ones((_FUSED_ROWS, 1), jnp.int32)
    off = 0
    for f, size in enumerate(_SIZES):
        in_f = jnp.logical_and(iota >= off, iota < off + size)
        mask = jnp.where(in_f, ((1 << _BITS[f]) - 1) << _SHIFTS[f], mask)
        tgt = jnp.where(in_f, (iota - off) << _SHIFTS[f], tgt)
        off += size
    return mask, tgt


def _make_body(n_groups):
    def _body(code_ref, tbl_ref, out_ref):
        tbl = tbl_ref[...]
        mask, tgt = _sublane_tables()

        def group(g, carry):
            g0 = pl.multiple_of(g, 1)
            code = code_ref[pl.ds(g0, 1), :]
            hot_t = ((code & mask) == tgt).astype(jnp.bfloat16)
            out_ref[pl.ds(g0 * _LANES, _LANES), :] = jax.lax.dot_general(
                hot_t, tbl, (((0,), (0,)), ((), ())),
                preferred_element_type=jnp.float32)
            return carry

        jax.lax.fori_loop(0, n_groups, group, None, unroll=True)

    return _body


def kernel(inputs, fused_table, *, tile_rows=8192):
    B, L, F = inputs.shape
    assert F == len(_SIZES)
    k_rows, d_model = fused_table.shape
    assert k_rows == _FUSED_ROWS

    N = B * L
    assert N % _LANES == 0
    x = inputs.astype(jnp.int32).reshape(N // _LANES, _LANES, F)
    code = (x[..., 0] | (x[..., 1] << _SHIFTS[1]) | (x[..., 2] << _SHIFTS[2])
            | (x[..., 3] << _SHIFTS[3]) | (x[..., 4] << _SHIFTS[4]))
    tbl16 = fused_table.astype(jnp.bfloat16)

    tile_rows = min(tile_rows, N)
    assert tile_rows % _LANES == 0 and N % tile_rows == 0
    n_groups = tile_rows // _LANES
    steps = N // tile_rows

    dp = ((d_model + 127) // 128) * 128
    vmem_need = (2 * n_groups * _LANES * 4
                 + 2 * tile_rows * dp * 4
                 + 2 * _FUSED_ROWS * dp * 2
                 + (4 << 20))
    vmem_limit = int(min(56 << 20, max(int(vmem_need * 1.2), 16 << 20)))

    out = pl.pallas_call(
        _make_body(n_groups),
        out_shape=jax.ShapeDtypeStruct((N, d_model), jnp.float32),
        grid=(steps,),
        in_specs=[
            pl.BlockSpec((n_groups, _LANES), lambda i: (i, 0)),
            pl.BlockSpec((_FUSED_ROWS, d_model), lambda i: (0, 0)),
        ],
        out_specs=pl.BlockSpec((tile_rows, d_model), lambda i: (i, 0)),
        compiler_params=pltpu.CompilerParams(
            dimension_semantics=("arbitrary",),
            vmem_limit_bytes=vmem_limit,
        ),
    )(code, tbl16)

    return out.reshape(B, L, d_model)
